# Initial kernel scaffold; baseline (speedup 1.0000x reference)
import jax
import jax.numpy as jnp
from jax.experimental import pallas as pl


def kernel(x, edge_index, edge_index_u, edge_index_v, params):
    raise NotImplementedError("write your pallas kernel here")


# speedup vs baseline: 5.1423x; 5.1423x over previous
import functools

import jax
import jax.numpy as jnp
from jax import lax
from jax.experimental import pallas as pl
from jax.experimental.pallas import tpu as pltpu
from jax.experimental.pallas import tpu_sc as plsc

_N = 50000
_NU = 25000
_E = 800000
_NP = 51200
_K = 128
_GB = 4
_NSUB = 16
_NCORE = 2
_BN = 3200


def _ceil_to(x, m):
    return (x + m - 1) // m * m


_EP = _ceil_to(_E, _K * _GB * _NSUB)
_EP2 = _ceil_to(_E // 2, _K * _GB * _NSUB)


@functools.lru_cache(None)
def _make_seg_sum(nb, w, ep):
    ch = nb // _NCORE
    cpt = ep // _K // _NSUB
    ngrp = cpt // _GB
    nrr = _NP // _NSUB
    mesh = plsc.VectorSubcoreMesh(core_axis_name="c", subcore_axis_name="s")

    def body(vals, srcs, dsts, out, acc, sidx, didx, rows, gsem):
        c = lax.axis_index("c")
        s = lax.axis_index("s")
        for bi in range(ch):
            b = c * ch + bi
            row0 = b * _NP
            pltpu.sync_copy(vals.at[pl.ds(row0 + s * nrr, nrr)],
                            acc.at[pl.ds(s * nrr, nrr)])
            plsc.subcore_barrier()

            def grp(g, carry):
                base = s * cpt + g * _GB
                pltpu.sync_copy(srcs.at[b, pl.ds(base, _GB)], sidx)
                pltpu.sync_copy(dsts.at[b, pl.ds(base, _GB)], didx)
                descs = [
                    pltpu.async_copy(vals.at[sidx.at[j]], rows.at[j], gsem)
                    for j in range(_GB)
                ]
                for j in range(_GB):
                    descs[j].wait()
                    pltpu.sync_copy(rows.at[j], acc.at[didx.at[j]], add=True)
                return carry

            lax.fori_loop(0, ngrp, grp, 0)
            plsc.subcore_barrier()
            pltpu.sync_copy(acc.at[pl.ds(s * nrr, nrr)],
                            out.at[pl.ds(row0 + s * nrr, nrr)])
            plsc.subcore_barrier()

    return pl.kernel(
        body,
        out_type=jax.ShapeDtypeStruct((nb * _NP, w), jnp.float32),
        mesh=mesh,
        scratch_types=[
            pltpu.VMEM_SHARED((_NP, w), jnp.float32),
            pltpu.VMEM((_GB, _K), jnp.int32),
            pltpu.VMEM((_GB, _K), jnp.int32),
            pltpu.VMEM((_GB, _K, w), jnp.float32),
            pltpu.SemaphoreType.DMA,
        ],
        compiler_params=pltpu.CompilerParams(use_tc_tiling_on_sc=False),
    )


def _tc_run(body, row_ins, bcast_ins, out_cols):
    grid = (_NP // _BN,)
    in_specs = (
        [pl.BlockSpec((_BN, a.shape[1]), lambda i: (i, 0)) for a in row_ins]
        + [pl.BlockSpec(w.shape, lambda i: (0, 0)) for w in bcast_ins]
    )
    out_specs = [pl.BlockSpec((_BN, c), lambda i: (i, 0)) for c in out_cols]
    out_shape = [jax.ShapeDtypeStruct((_NP, c), jnp.float32) for c in out_cols]
    res = pl.pallas_call(
        body, grid=grid, in_specs=in_specs, out_specs=out_specs,
        out_shape=out_shape,
    )(*row_ins, *bcast_ins)
    return res


def _row_ids(cols):
    return (pl.program_id(0) * _BN
            + lax.broadcasted_iota(jnp.int32, (_BN, cols), 0))


def _t1_body(x_r, du_r, dv_r, da_r, db_r, diu_r, div_r, diuv_r, xsu_r, xsv_r):
    x = x_r[...]
    diu = 1.0 / jnp.sqrt(du_r[:, 0:1])
    div = 1.0 / jnp.sqrt(dv_r[:, 0:1])
    diuv = 1.0 / jnp.sqrt(da_r[:, 0:1] + db_r[:, 0:1] - 1.0)
    diu_r[...] = jnp.broadcast_to(diu, (_BN, 16))
    div_r[...] = jnp.broadcast_to(div, (_BN, 16))
    diuv_r[...] = jnp.broadcast_to(diuv, (_BN, 16))
    xsu_r[...] = jnp.broadcast_to(x * diu, (_BN, 16))
    xsv_r[...] = jnp.broadcast_to(x * div, (_BN, 16))


def _t2_body(aggu_r, aggv_r, diu_r, div_r, diuv_r, wu_r, bu_r, wv_r, bv_r,
             xs_r):
    yu = bu_r[...] + (diu_r[:, 0:1] * aggu_r[:, 0:1]) * wu_r[...]
    yv = bv_r[...] + (div_r[:, 0:1] * aggv_r[:, 0:1]) * wv_r[...]
    y = jnp.where(_row_ids(64) < _NU, yu, yv)
    xs_r[...] = diuv_r[:, 0:1] * y


def _t3_body(a0_r, a1_r, diu_r, div_r, diuv_r, wuv_r, buv_r, wu2_r, wv2_r,
             xsu_r, xsv_r):
    agg = jnp.concatenate([a0_r[...], a1_r[...]], axis=1)
    x2 = jax.nn.relu(
        buv_r[...]
        + jnp.dot(diuv_r[:, 0:1] * agg, wuv_r[...],
                  preferred_element_type=jnp.float32))
    xsu_r[...] = diu_r[:, 0:1] * jnp.dot(x2, wu2_r[...],
                                         preferred_element_type=jnp.float32)
    xsv_r[...] = div_r[:, 0:1] * jnp.dot(x2, wv2_r[...],
                                         preferred_element_type=jnp.float32)


def _t4_body(a0_r, a1_r, diu_r, div_r, diuv_r, bu_r, bv_r, xs_r):
    outu = bu_r[...] + diu_r[:, 0:1] * a0_r[...]
    outv = bv_r[...] + div_r[:, 0:1] * a1_r[...]
    y = jnp.where(_row_ids(32) < _NU, outu, outv)
    xs_r[...] = diuv_r[:, 0:1] * y


def _t5_body(a0_r, a1_r, diu_r, div_r, diuv_r, wuv_r, buv_r, xsu_r, xsv_r):
    agg = jnp.concatenate([a0_r[...], a1_r[...]], axis=1)
    x4 = jax.nn.relu(
        buv_r[...]
        + jnp.dot(diuv_r[:, 0:1] * agg, wuv_r[...],
                  preferred_element_type=jnp.float32))
    xsu_r[...] = diu_r[:, 0:1] * x4
    xsv_r[...] = div_r[:, 0:1] * x4


def _t6_body(a0_r, a1_r, diu_r, div_r, diuv_r, wu_r, bu_r, wv_r, bv_r, xs_r):
    tu = bu_r[...] + jnp.dot(diu_r[:, 0:1] * a0_r[...], wu_r[...],
                             preferred_element_type=jnp.float32)
    tv = bv_r[...] + jnp.dot(div_r[:, 0:1] * a1_r[...], wv_r[...],
                             preferred_element_type=jnp.float32)
    t = jnp.where(_row_ids(128) < _NU, tu, tv)
    xs_r[...] = diuv_r[:, 0:1] * t


def _t7_body(a0_r, a1_r, a2_r, a3_r, diuv_r, wuv_r, buv_r, out_r):
    agg = jnp.concatenate([a0_r[...], a1_r[...], a2_r[...], a3_r[...]],
                          axis=1)
    out_r[...] = buv_r[...] + jnp.dot(
        diuv_r[:, 0:1] * agg, wuv_r[...], preferred_element_type=jnp.float32)


def kernel(x, edge_index, edge_index_u, edge_index_v, params):
    f32 = jnp.float32
    ((Wu1, bu1, Wv1, bv1, Wuv1, buv1),
     (Wu2, bu2, Wv2, bv2, Wuv2, buv2),
     (Wu3, bu3, Wv3, bv3, Wuv3, buv3)) = params

    src_uv, dst_uv = edge_index[0], edge_index[1]
    src_u, dst_u = edge_index_u[0], edge_index_u[1]
    src_v, dst_v = edge_index_v[0], edge_index_v[1]

    def pad_e(a, val, ep):
        return jnp.pad(a, (0, ep - a.shape[0]), constant_values=val)

    def mk_idx(blocks, ep):
        return jnp.stack(blocks).reshape(len(blocks), ep // _K, _K)

    spu = pad_e(src_u, 0, _EP)
    spv = pad_e(src_v, 0, _EP)
    spuv = pad_e(src_uv, 0, _EP)
    dpu = pad_e(dst_u, _N, _EP)
    dpv = pad_e(dst_v, _N, _EP)
    dpuv = pad_e(dst_uv, _N, _EP)

    src_ace = mk_idx([spu, spv + _NP], _EP)
    dst_ace = mk_idx([dpu, dpv], _EP)
    src_bd = mk_idx([spuv, spuv + _NP], _EP)
    dst_bd = mk_idx([dpuv, dpuv], _EP)
    src_f = mk_idx([spuv + i * _NP for i in range(4)], _EP)
    dst_f = mk_idx([dpuv] * 4, _EP)
    zsrc2 = jnp.zeros((2, _EP // _K, _K), jnp.int32)
    zsrc2h = jnp.zeros((2, _EP2 // _K, _K), jnp.int32)
    dst_h2 = mk_idx([pad_e(dst_uv[:_E // 2], _N, _EP2),
                     pad_e(dst_uv[_E // 2:], _N, _EP2)], _EP2)

    seg16 = _make_seg_sum(2, 16, _EP)
    seg16h = _make_seg_sum(2, 16, _EP2)
    seg32 = _make_seg_sum(2, 32, _EP)
    seg32x4 = _make_seg_sum(4, 32, _EP)

    ones2 = jnp.ones((2 * _NP, 16), f32)
    h1 = seg16(ones2, zsrc2, dst_ace)
    h2 = seg16h(ones2, zsrc2h, dst_h2)

    xpad = jnp.pad(x, ((0, _NP - _N), (0, 0)))
    diu, div, diuv, xsu, xsv = _tc_run(
        _t1_body, [xpad, h1[:_NP], h1[_NP:], h2[:_NP], h2[_NP:]], [],
        [16, 16, 16, 16, 16])

    agg_a = seg16(jnp.concatenate([xsu, xsv]), src_ace, dst_ace)
    xs_b = _tc_run(
        _t2_body, [agg_a[:_NP], agg_a[_NP:], diu, div, diuv],
        [Wu1.reshape(1, 64), bu1.reshape(1, 64),
         Wv1.reshape(1, 64), bv1.reshape(1, 64)], [64])[0]
    agg_b = seg32(jnp.concatenate([xs_b[:, :32], xs_b[:, 32:]]),
                  src_bd, dst_bd)

    xsu2, xsv2 = _tc_run(
        _t3_body, [agg_b[:_NP], agg_b[_NP:], diu, div, diuv],
        [Wuv1, buv1.reshape(1, 64), Wu2, Wv2], [32, 32])
    agg_c = seg32(jnp.concatenate([xsu2, xsv2]), src_ace, dst_ace)
    xs_d = _tc_run(
        _t4_body, [agg_c[:_NP], agg_c[_NP:], diu, div, diuv],
        [bu2.reshape(1, 32), bv2.reshape(1, 32)], [32])[0]
    agg_d = seg16(jnp.concatenate([xs_d[:, :16], xs_d[:, 16:]]),
                  src_bd, dst_bd)

    xsu3, xsv3 = _tc_run(
        _t5_body, [agg_d[:_NP], agg_d[_NP:], diu, div, diuv],
        [Wuv2, buv2.reshape(1, 32)], [32, 32])
    agg_e = seg32(jnp.concatenate([xsu3, xsv3]), src_ace, dst_ace)
    xs_f = _tc_run(
        _t6_body, [agg_e[:_NP], agg_e[_NP:], diu, div, diuv],
        [Wu3, bu3.reshape(1, 128), Wv3, bv3.reshape(1, 128)], [128])[0]
    agg_f = seg32x4(
        jnp.concatenate([xs_f[:, 0:32], xs_f[:, 32:64],
                         xs_f[:, 64:96], xs_f[:, 96:128]]), src_f, dst_f)
    out = _tc_run(
        _t7_body,
        [agg_f[:_NP], agg_f[_NP:2 * _NP], agg_f[2 * _NP:3 * _NP],
         agg_f[3 * _NP:], diuv],
        [Wuv3, buv3.reshape(1, 128)], [128])[0]
    return out[:_N]


# speedup vs baseline: 19.4900x; 3.7902x over previous
import functools

import jax
import jax.numpy as jnp
from jax import lax
from jax.experimental import pallas as pl
from jax.experimental.pallas import tpu as pltpu
from jax.experimental.pallas import tpu_sc as plsc

_N = 50000
_NU = 25000
_E = 800000
_NP = 51200
_K = 128
_GB = 4
_NSUB = 16
_NCORE = 2
_BN = 3200


def _ceil_to(x, m):
    return (x + m - 1) // m * m


_EP = _ceil_to(_E, _K * _GB * _NSUB)
_EP2 = _ceil_to(_E // 2, _K * _GB * _NSUB)


@functools.lru_cache(None)
def _make_seg_sum(nb, w, ep, gather=True):
    ch = nb // _NCORE
    cpt = ep // _K // _NSUB
    ngrp = cpt // _GB
    nrr = _NP // _NSUB
    mesh = plsc.VectorSubcoreMesh(core_axis_name="c", subcore_axis_name="s")

    def body(vals, srcs, dsts, out, acc, sidx, didx, rows, gsem):
        c = lax.axis_index("c")
        s = lax.axis_index("s")
        if not gather:
            for j in range(_GB):
                pltpu.sync_copy(vals.at[pl.ds(0, _K)], rows.at[j])
        for bi in range(ch):
            b = c * ch + bi
            row0 = b * _NP
            pltpu.sync_copy(vals.at[pl.ds(row0 + s * nrr, nrr)],
                            acc.at[pl.ds(s * nrr, nrr)])
            plsc.subcore_barrier()

            def grp(g, carry):
                base = s * cpt + g * _GB
                pltpu.sync_copy(dsts.at[b, pl.ds(base, _GB)], didx)
                if gather:
                    pltpu.sync_copy(srcs.at[b, pl.ds(base, _GB)], sidx)
                    descs = [
                        pltpu.async_copy(vals.at[sidx.at[j]], rows.at[j], gsem)
                        for j in range(_GB)
                    ]
                    for j in range(_GB):
                        descs[j].wait()
                        pltpu.sync_copy(rows.at[j], acc.at[didx.at[j]],
                                        add=True)
                else:
                    for j in range(_GB):
                        pltpu.sync_copy(rows.at[j], acc.at[didx.at[j]],
                                        add=True)
                return carry

            lax.fori_loop(0, ngrp, grp, 0)
            plsc.subcore_barrier()
            pltpu.sync_copy(acc.at[pl.ds(s * nrr, nrr)],
                            out.at[pl.ds(row0 + s * nrr, nrr)])
            plsc.subcore_barrier()

    return pl.kernel(
        body,
        out_type=jax.ShapeDtypeStruct((nb * _NP, w), jnp.float32),
        mesh=mesh,
        scratch_types=[
            pltpu.VMEM_SHARED((_NP, w), jnp.float32),
            pltpu.VMEM((_GB, _K), jnp.int32),
            pltpu.VMEM((_GB, _K), jnp.int32),
            pltpu.VMEM((_GB, _K, w), jnp.float32),
            pltpu.SemaphoreType.DMA,
        ],
        compiler_params=pltpu.CompilerParams(use_tc_tiling_on_sc=False),
    )


def _tc_run(body, row_ins, bcast_ins, out_cols):
    grid = (_NP // _BN,)
    in_specs = (
        [pl.BlockSpec((_BN, a.shape[1]), lambda i: (i, 0)) for a in row_ins]
        + [pl.BlockSpec(w.shape, lambda i: (0, 0)) for w in bcast_ins]
    )
    out_specs = [pl.BlockSpec((_BN, c), lambda i: (i, 0)) for c in out_cols]
    out_shape = [jax.ShapeDtypeStruct((_NP, c), jnp.float32) for c in out_cols]
    res = pl.pallas_call(
        body, grid=grid, in_specs=in_specs, out_specs=out_specs,
        out_shape=out_shape,
    )(*row_ins, *bcast_ins)
    return res


def _row_ids(cols):
    return (pl.program_id(0) * _BN
            + lax.broadcasted_iota(jnp.int32, (_BN, cols), 0))


def _t1_body(x_r, du_r, dv_r, da_r, db_r, diu_r, div_r, diuv_r, xsu_r, xsv_r):
    x = x_r[...]
    diu = 1.0 / jnp.sqrt(du_r[:, 0:1])
    div = 1.0 / jnp.sqrt(dv_r[:, 0:1])
    diuv = 1.0 / jnp.sqrt(da_r[:, 0:1] + db_r[:, 0:1] - 1.0)
    diu_r[...] = jnp.broadcast_to(diu, (_BN, 16))
    div_r[...] = jnp.broadcast_to(div, (_BN, 16))
    diuv_r[...] = jnp.broadcast_to(diuv, (_BN, 16))
    xsu_r[...] = jnp.broadcast_to(x * diu, (_BN, 16))
    xsv_r[...] = jnp.broadcast_to(x * div, (_BN, 16))


def _t2_body(aggu_r, aggv_r, diu_r, div_r, diuv_r, wu_r, bu_r, wv_r, bv_r,
             xs_r):
    yu = bu_r[...] + (diu_r[:, 0:1] * aggu_r[:, 0:1]) * wu_r[...]
    yv = bv_r[...] + (div_r[:, 0:1] * aggv_r[:, 0:1]) * wv_r[...]
    y = jnp.where(_row_ids(64) < _NU, yu, yv)
    xs_r[...] = diuv_r[:, 0:1] * y


def _t3_body(a0_r, a1_r, diu_r, div_r, diuv_r, wuv_r, buv_r, wu2_r, wv2_r,
             xsu_r, xsv_r):
    agg = jnp.concatenate([a0_r[...], a1_r[...]], axis=1)
    x2 = jax.nn.relu(
        buv_r[...]
        + jnp.dot(diuv_r[:, 0:1] * agg, wuv_r[...],
                  preferred_element_type=jnp.float32))
    xsu_r[...] = diu_r[:, 0:1] * jnp.dot(x2, wu2_r[...],
                                         preferred_element_type=jnp.float32)
    xsv_r[...] = div_r[:, 0:1] * jnp.dot(x2, wv2_r[...],
                                         preferred_element_type=jnp.float32)


def _t4_body(a0_r, a1_r, diu_r, div_r, diuv_r, bu_r, bv_r, xs_r):
    outu = bu_r[...] + diu_r[:, 0:1] * a0_r[...]
    outv = bv_r[...] + div_r[:, 0:1] * a1_r[...]
    y = jnp.where(_row_ids(32) < _NU, outu, outv)
    xs_r[...] = diuv_r[:, 0:1] * y


def _t5_body(a0_r, a1_r, diu_r, div_r, diuv_r, wuv_r, buv_r, xsu_r, xsv_r):
    agg = jnp.concatenate([a0_r[...], a1_r[...]], axis=1)
    x4 = jax.nn.relu(
        buv_r[...]
        + jnp.dot(diuv_r[:, 0:1] * agg, wuv_r[...],
                  preferred_element_type=jnp.float32))
    xsu_r[...] = diu_r[:, 0:1] * x4
    xsv_r[...] = div_r[:, 0:1] * x4


def _t6_body(a0_r, a1_r, diu_r, div_r, diuv_r, wu_r, bu_r, wv_r, bv_r, xs_r):
    tu = bu_r[...] + jnp.dot(diu_r[:, 0:1] * a0_r[...], wu_r[...],
                             preferred_element_type=jnp.float32)
    tv = bv_r[...] + jnp.dot(div_r[:, 0:1] * a1_r[...], wv_r[...],
                             preferred_element_type=jnp.float32)
    t = jnp.where(_row_ids(128) < _NU, tu, tv)
    xs_r[...] = diuv_r[:, 0:1] * t


def _t7_body(a0_r, a1_r, a2_r, a3_r, diuv_r, wuv_r, buv_r, out_r):
    agg = jnp.concatenate([a0_r[...], a1_r[...], a2_r[...], a3_r[...]],
                          axis=1)
    out_r[...] = buv_r[...] + jnp.dot(
        diuv_r[:, 0:1] * agg, wuv_r[...], preferred_element_type=jnp.float32)


def kernel(x, edge_index, edge_index_u, edge_index_v, params):
    f32 = jnp.float32
    ((Wu1, bu1, Wv1, bv1, Wuv1, buv1),
     (Wu2, bu2, Wv2, bv2, Wuv2, buv2),
     (Wu3, bu3, Wv3, bv3, Wuv3, buv3)) = params

    src_uv, dst_uv = edge_index[0], edge_index[1]
    src_u, dst_u = edge_index_u[0], edge_index_u[1]
    src_v, dst_v = edge_index_v[0], edge_index_v[1]

    def pad_e(a, val, ep):
        return jnp.pad(a, (0, ep - a.shape[0]), constant_values=val)

    def mk_idx(blocks, ep):
        return jnp.stack(blocks).reshape(len(blocks), ep // _K, _K)

    spu = pad_e(src_u, 0, _EP)
    spv = pad_e(src_v, 0, _EP)
    spuv = pad_e(src_uv, 0, _EP)
    dpu = pad_e(dst_u, _N, _EP)
    dpv = pad_e(dst_v, _N, _EP)
    dpuv = pad_e(dst_uv, _N, _EP)

    src_ace = mk_idx([spu, spv + _NP], _EP)
    dst_ace = mk_idx([dpu, dpv], _EP)
    src_bd = mk_idx([spuv, spuv + _NP], _EP)
    dst_bd = mk_idx([dpuv, dpuv], _EP)
    src_f = mk_idx([spuv + i * _NP for i in range(4)], _EP)
    dst_f = mk_idx([dpuv] * 4, _EP)
    dst_h2 = mk_idx([pad_e(dst_uv[:_E // 2], _N, _EP2),
                     pad_e(dst_uv[_E // 2:], _N, _EP2)], _EP2)

    seg16 = _make_seg_sum(2, 16, _EP)
    seg32 = _make_seg_sum(2, 32, _EP)
    seg32x4 = _make_seg_sum(4, 32, _EP)
    cnt16 = _make_seg_sum(2, 16, _EP, gather=False)
    cnt16h = _make_seg_sum(2, 16, _EP2, gather=False)

    ones2 = jnp.ones((2 * _NP, 16), f32)
    h1 = cnt16(ones2, dst_ace, dst_ace)
    h2 = cnt16h(ones2, dst_h2, dst_h2)

    xpad = jnp.pad(x, ((0, _NP - _N), (0, 0)))
    diu, div, diuv, xsu, xsv = _tc_run(
        _t1_body, [xpad, h1[:_NP], h1[_NP:], h2[:_NP], h2[_NP:]], [],
        [16, 16, 16, 16, 16])

    agg_a = seg16(jnp.concatenate([xsu, xsv]), src_ace, dst_ace)
    xs_b = _tc_run(
        _t2_body, [agg_a[:_NP], agg_a[_NP:], diu, div, diuv],
        [Wu1.reshape(1, 64), bu1.reshape(1, 64),
         Wv1.reshape(1, 64), bv1.reshape(1, 64)], [64])[0]
    agg_b = seg32(jnp.concatenate([xs_b[:, :32], xs_b[:, 32:]]),
                  src_bd, dst_bd)

    xsu2, xsv2 = _tc_run(
        _t3_body, [agg_b[:_NP], agg_b[_NP:], diu, div, diuv],
        [Wuv1, buv1.reshape(1, 64), Wu2, Wv2], [32, 32])
    agg_c = seg32(jnp.concatenate([xsu2, xsv2]), src_ace, dst_ace)
    xs_d = _tc_run(
        _t4_body, [agg_c[:_NP], agg_c[_NP:], diu, div, diuv],
        [bu2.reshape(1, 32), bv2.reshape(1, 32)], [32])[0]
    agg_d = seg16(jnp.concatenate([xs_d[:, :16], xs_d[:, 16:]]),
                  src_bd, dst_bd)

    xsu3, xsv3 = _tc_run(
        _t5_body, [agg_d[:_NP], agg_d[_NP:], diu, div, diuv],
        [Wuv2, buv2.reshape(1, 32)], [32, 32])
    agg_e = seg32(jnp.concatenate([xsu3, xsv3]), src_ace, dst_ace)
    xs_f = _tc_run(
        _t6_body, [agg_e[:_NP], agg_e[_NP:], diu, div, diuv],
        [Wu3, bu3.reshape(1, 128), Wv3, bv3.reshape(1, 128)], [128])[0]
    agg_f = seg32x4(
        jnp.concatenate([xs_f[:, 0:32], xs_f[:, 32:64],
                         xs_f[:, 64:96], xs_f[:, 96:128]]), src_f, dst_f)
    out = _tc_run(
        _t7_body,
        [agg_f[:_NP], agg_f[_NP:2 * _NP], agg_f[2 * _NP:3 * _NP],
         agg_f[3 * _NP:], diuv],
        [Wuv3, buv3.reshape(1, 128)], [128])[0]
    return out[:_N]


# speedup vs baseline: 22.8552x; 1.1727x over previous
import functools

import jax
import jax.numpy as jnp
from jax import lax
from jax.experimental import pallas as pl
from jax.experimental.pallas import tpu as pltpu
from jax.experimental.pallas import tpu_sc as plsc

_N = 50000
_NU = 25000
_E = 800000
_NP = 51200
_K = 128
_GB = 4
_NSUB = 16
_NCORE = 2
_BN = 3200


def _ceil_to(x, m):
    return (x + m - 1) // m * m


_EP = _ceil_to(_E, _K * _GB * _NSUB)
_EP2 = _ceil_to(_E // 2, _K * _GB * _NSUB)


@functools.lru_cache(None)
def _make_seg_sum(nb, w, ep, gather=True):
    gb = _GB if w == 16 else 2
    ch = nb // _NCORE
    cpt = ep // _K // _NSUB
    ngrp = cpt // gb
    nrr = _NP // _NSUB
    mesh = plsc.VectorSubcoreMesh(core_axis_name="c", subcore_axis_name="s")

    def body(vals, sd, out, acc, sdix, rows, gsem, ssem):
        c = lax.axis_index("c")
        s = lax.axis_index("s")

        def load(bb, base0, g, slot):
            pltpu.sync_copy(sd.at[bb, pl.ds(base0 + g * gb, gb)],
                            sdix.at[slot])

        def fire_gathers(slot):
            for j in range(gb):
                pltpu.async_copy(vals.at[sdix.at[slot, j, 0]],
                                 rows.at[slot, j], gsem)

        def wait_gathers(slot):
            for j in range(gb):
                pltpu.make_async_copy(vals.at[sdix.at[slot, j, 0]],
                                      rows.at[slot, j], gsem).wait()

        def scatters(slot):
            descs = [
                pltpu.async_copy(rows.at[slot, j], acc.at[sdix.at[slot, j, 1]],
                                 ssem, add=True)
                for j in range(gb)
            ]
            for d in descs:
                d.wait()

        def step(bb, base0, g, p, prefetch, gnext):
            pn = 1 - p
            if prefetch:
                load(bb, base0, gnext, pn)
            wait_gathers(p)
            if prefetch:
                fire_gathers(pn)
            scatters(p)

        if not gather:
            for j in range(gb):
                pltpu.sync_copy(vals.at[pl.ds(0, _K)], rows.at[0, j])
        for bi in range(ch):
            b = c * ch + bi
            row0 = b * _NP
            base0 = s * cpt
            pltpu.sync_copy(vals.at[pl.ds(row0 + s * nrr, nrr)],
                            acc.at[pl.ds(s * nrr, nrr)])
            plsc.subcore_barrier()

            if gather:
                load(b, base0, 0, 0)
                fire_gathers(0)
                m = (ngrp - 1) // 2

                def dbl(t, carry):
                    g = t * 2
                    step(b, base0, g, 0, True, g + 1)
                    step(b, base0, g + 1, 1, True, g + 2)
                    return carry

                lax.fori_loop(0, m, dbl, 0)
                g0 = 2 * m
                if (ngrp - 1) % 2 == 1:
                    step(b, base0, g0, 0, True, g0 + 1)
                    step(b, base0, g0 + 1, 1, False, 0)
                else:
                    step(b, base0, g0, 0, False, 0)
            else:

                def grp(g, carry):
                    load(b, base0, g, 0)
                    scatters(0)
                    return carry

                lax.fori_loop(0, ngrp, grp, 0)
            plsc.subcore_barrier()
            pltpu.sync_copy(acc.at[pl.ds(s * nrr, nrr)],
                            out.at[pl.ds(row0 + s * nrr, nrr)])
            plsc.subcore_barrier()

    return pl.kernel(
        body,
        out_type=jax.ShapeDtypeStruct((nb * _NP, w), jnp.float32),
        mesh=mesh,
        scratch_types=[
            pltpu.VMEM_SHARED((_NP, w), jnp.float32),
            pltpu.VMEM((2, gb, 2, _K), jnp.int32),
            pltpu.VMEM((2, gb, _K, w), jnp.float32),
            pltpu.SemaphoreType.DMA,
            pltpu.SemaphoreType.DMA,
        ],
        compiler_params=pltpu.CompilerParams(use_tc_tiling_on_sc=False),
    )


def _tc_run(body, row_ins, bcast_ins, out_cols):
    grid = (_NP // _BN,)
    in_specs = (
        [pl.BlockSpec((_BN, a.shape[1]), lambda i: (i, 0)) for a in row_ins]
        + [pl.BlockSpec(w.shape, lambda i: (0, 0)) for w in bcast_ins]
    )
    out_specs = [pl.BlockSpec((_BN, c), lambda i: (i, 0)) for c in out_cols]
    out_shape = [jax.ShapeDtypeStruct((_NP, c), jnp.float32) for c in out_cols]
    res = pl.pallas_call(
        body, grid=grid, in_specs=in_specs, out_specs=out_specs,
        out_shape=out_shape,
    )(*row_ins, *bcast_ins)
    return res


def _row_ids(cols):
    return (pl.program_id(0) * _BN
            + lax.broadcasted_iota(jnp.int32, (_BN, cols), 0))


def _t1_body(x_r, du_r, dv_r, da_r, db_r, diu_r, div_r, diuv_r, xsu_r, xsv_r):
    x = x_r[...]
    diu = 1.0 / jnp.sqrt(du_r[:, 0:1])
    div = 1.0 / jnp.sqrt(dv_r[:, 0:1])
    diuv = 1.0 / jnp.sqrt(da_r[:, 0:1] + db_r[:, 0:1] - 1.0)
    diu_r[...] = jnp.broadcast_to(diu, (_BN, 16))
    div_r[...] = jnp.broadcast_to(div, (_BN, 16))
    diuv_r[...] = jnp.broadcast_to(diuv, (_BN, 16))
    xsu_r[...] = jnp.broadcast_to(x * diu, (_BN, 16))
    xsv_r[...] = jnp.broadcast_to(x * div, (_BN, 16))


def _t2_body(aggu_r, aggv_r, diu_r, div_r, diuv_r, wu_r, bu_r, wv_r, bv_r,
             xs_r):
    yu = bu_r[...] + (diu_r[:, 0:1] * aggu_r[:, 0:1]) * wu_r[...]
    yv = bv_r[...] + (div_r[:, 0:1] * aggv_r[:, 0:1]) * wv_r[...]
    y = jnp.where(_row_ids(64) < _NU, yu, yv)
    xs_r[...] = diuv_r[:, 0:1] * y


def _t3_body(a0_r, a1_r, diu_r, div_r, diuv_r, wuv_r, buv_r, wu2_r, wv2_r,
             xsu_r, xsv_r):
    agg = jnp.concatenate([a0_r[...], a1_r[...]], axis=1)
    x2 = jax.nn.relu(
        buv_r[...]
        + jnp.dot(diuv_r[:, 0:1] * agg, wuv_r[...],
                  preferred_element_type=jnp.float32))
    xsu_r[...] = diu_r[:, 0:1] * jnp.dot(x2, wu2_r[...],
                                         preferred_element_type=jnp.float32)
    xsv_r[...] = div_r[:, 0:1] * jnp.dot(x2, wv2_r[...],
                                         preferred_element_type=jnp.float32)


def _t4_body(a0_r, a1_r, diu_r, div_r, diuv_r, bu_r, bv_r, xs_r):
    outu = bu_r[...] + diu_r[:, 0:1] * a0_r[...]
    outv = bv_r[...] + div_r[:, 0:1] * a1_r[...]
    y = jnp.where(_row_ids(32) < _NU, outu, outv)
    xs_r[...] = diuv_r[:, 0:1] * y


def _t5_body(a0_r, a1_r, diu_r, div_r, diuv_r, wuv_r, buv_r, xsu_r, xsv_r):
    agg = jnp.concatenate([a0_r[...], a1_r[...]], axis=1)
    x4 = jax.nn.relu(
        buv_r[...]
        + jnp.dot(diuv_r[:, 0:1] * agg, wuv_r[...],
                  preferred_element_type=jnp.float32))
    xsu_r[...] = diu_r[:, 0:1] * x4
    xsv_r[...] = div_r[:, 0:1] * x4


def _t6_body(a0_r, a1_r, diu_r, div_r, diuv_r, wu_r, bu_r, wv_r, bv_r, xs_r):
    tu = bu_r[...] + jnp.dot(diu_r[:, 0:1] * a0_r[...], wu_r[...],
                             preferred_element_type=jnp.float32)
    tv = bv_r[...] + jnp.dot(div_r[:, 0:1] * a1_r[...], wv_r[...],
                             preferred_element_type=jnp.float32)
    t = jnp.where(_row_ids(128) < _NU, tu, tv)
    xs_r[...] = diuv_r[:, 0:1] * t


def _t7_body(a0_r, a1_r, a2_r, a3_r, diuv_r, wuv_r, buv_r, out_r):
    agg = jnp.concatenate([a0_r[...], a1_r[...], a2_r[...], a3_r[...]],
                          axis=1)
    out_r[...] = buv_r[...] + jnp.dot(
        diuv_r[:, 0:1] * agg, wuv_r[...], preferred_element_type=jnp.float32)


def kernel(x, edge_index, edge_index_u, edge_index_v, params):
    f32 = jnp.float32
    ((Wu1, bu1, Wv1, bv1, Wuv1, buv1),
     (Wu2, bu2, Wv2, bv2, Wuv2, buv2),
     (Wu3, bu3, Wv3, bv3, Wuv3, buv3)) = params

    src_uv, dst_uv = edge_index[0], edge_index[1]
    src_u, dst_u = edge_index_u[0], edge_index_u[1]
    src_v, dst_v = edge_index_v[0], edge_index_v[1]

    def pad_e(a, val, ep):
        return jnp.pad(a, (0, ep - a.shape[0]), constant_values=val)

    def mk_sd(src_blocks, dst_blocks, ep):
        srcs = jnp.stack(src_blocks).reshape(len(src_blocks), ep // _K, _K)
        dsts = jnp.stack(dst_blocks).reshape(len(dst_blocks), ep // _K, _K)
        return jnp.stack([srcs, dsts], axis=2)

    spu = pad_e(src_u, 0, _EP)
    spv = pad_e(src_v, 0, _EP)
    spuv = pad_e(src_uv, 0, _EP)
    dpu = pad_e(dst_u, _N, _EP)
    dpv = pad_e(dst_v, _N, _EP)
    dpuv = pad_e(dst_uv, _N, _EP)

    sd_ace = mk_sd([spu, spv + _NP], [dpu, dpv], _EP)
    sd_bd = mk_sd([spuv, spuv + _NP], [dpuv, dpuv], _EP)
    sd_f = mk_sd([spuv + i * _NP for i in range(4)], [dpuv] * 4, _EP)
    sd_h1 = mk_sd([dpu, dpv], [dpu, dpv], _EP)
    dh0 = pad_e(dst_uv[:_E // 2], _N, _EP2)
    dh1 = pad_e(dst_uv[_E // 2:], _N, _EP2)
    sd_h2 = mk_sd([dh0, dh1], [dh0, dh1], _EP2)

    seg16 = _make_seg_sum(2, 16, _EP)
    seg32 = _make_seg_sum(2, 32, _EP)
    seg32x4 = _make_seg_sum(4, 32, _EP)
    cnt16 = _make_seg_sum(2, 16, _EP, gather=False)
    cnt16h = _make_seg_sum(2, 16, _EP2, gather=False)

    ones2 = jnp.ones((2 * _NP, 16), f32)
    h1 = cnt16(ones2, sd_h1)
    h2 = cnt16h(ones2, sd_h2)

    xpad = jnp.pad(x, ((0, _NP - _N), (0, 0)))
    diu, div, diuv, xsu, xsv = _tc_run(
        _t1_body, [xpad, h1[:_NP], h1[_NP:], h2[:_NP], h2[_NP:]], [],
        [16, 16, 16, 16, 16])

    agg_a = seg16(jnp.concatenate([xsu, xsv]), sd_ace)
    xs_b = _tc_run(
        _t2_body, [agg_a[:_NP], agg_a[_NP:], diu, div, diuv],
        [Wu1.reshape(1, 64), bu1.reshape(1, 64),
         Wv1.reshape(1, 64), bv1.reshape(1, 64)], [64])[0]
    agg_b = seg32(jnp.concatenate([xs_b[:, :32], xs_b[:, 32:]]), sd_bd)

    xsu2, xsv2 = _tc_run(
        _t3_body, [agg_b[:_NP], agg_b[_NP:], diu, div, diuv],
        [Wuv1, buv1.reshape(1, 64), Wu2, Wv2], [32, 32])
    agg_c = seg32(jnp.concatenate([xsu2, xsv2]), sd_ace)
    xs_d = _tc_run(
        _t4_body, [agg_c[:_NP], agg_c[_NP:], diu, div, diuv],
        [bu2.reshape(1, 32), bv2.reshape(1, 32)], [32])[0]
    agg_d = seg16(jnp.concatenate([xs_d[:, :16], xs_d[:, 16:]]), sd_bd)

    xsu3, xsv3 = _tc_run(
        _t5_body, [agg_d[:_NP], agg_d[_NP:], diu, div, diuv],
        [Wuv2, buv2.reshape(1, 32)], [32, 32])
    agg_e = seg32(jnp.concatenate([xsu3, xsv3]), sd_ace)
    xs_f = _tc_run(
        _t6_body, [agg_e[:_NP], agg_e[_NP:], diu, div, diuv],
        [Wu3, bu3.reshape(1, 128), Wv3, bv3.reshape(1, 128)], [128])[0]
    agg_f = seg32x4(
        jnp.concatenate([xs_f[:, 0:32], xs_f[:, 32:64],
                         xs_f[:, 64:96], xs_f[:, 96:128]]), sd_f)
    out = _tc_run(
        _t7_body,
        [agg_f[:_NP], agg_f[_NP:2 * _NP], agg_f[2 * _NP:3 * _NP],
         agg_f[3 * _NP:], diuv],
        [Wuv3, buv3.reshape(1, 128)], [128])[0]
    return out[:_N]


# speedup vs baseline: 22.9272x; 1.0031x over previous
import functools

import jax
import jax.numpy as jnp
from jax import lax
from jax.experimental import pallas as pl
from jax.experimental.pallas import tpu as pltpu
from jax.experimental.pallas import tpu_sc as plsc

_N = 50000
_NU = 25000
_E = 800000
_NP = 51200
_K = 128
_GB = 4
_NSUB = 16
_NCORE = 2
_BN = 3200


def _ceil_to(x, m):
    return (x + m - 1) // m * m


_EP = _ceil_to(_E, _K * _GB * _NSUB)


@functools.lru_cache(None)
def _make_seg_sum(nb, w, ep, gather=True, shared=False, split=False):
    gb = _GB if w == 16 else 2
    ch = nb // _NCORE
    cpb = ep // _K // (2 if split else 1)
    cpt = cpb // _NSUB
    ngrp = cpt // gb
    nrr = _NP // _NSUB
    one_list = shared or split
    mesh = plsc.VectorSubcoreMesh(core_axis_name="c", subcore_axis_name="s")

    def body(vals, sd, out, acc, sdix, rows, gsem, ssem):
        c = lax.axis_index("c")
        s = lax.axis_index("s")

        def load(bb, base0, g, slot, row0):
            pltpu.sync_copy(sd.at[0 if one_list else bb,
                                  pl.ds(base0 + g * gb, gb)],
                            sdix.at[slot])
            if gather:
                for j in range(gb):
                    for t in range(_K // 16):
                        ix = pl.ds(t * 16, 16)
                        sdix[slot, j, 0, ix] = sdix[slot, j, 0, ix] + row0

        def fire_gathers(slot):
            for j in range(gb):
                pltpu.async_copy(vals.at[sdix.at[slot, j, 0]],
                                 rows.at[slot, j], gsem)

        def wait_gathers(slot):
            for j in range(gb):
                pltpu.make_async_copy(vals.at[sdix.at[slot, j, 0]],
                                      rows.at[slot, j], gsem).wait()

        def scatters(slot):
            descs = [
                pltpu.async_copy(rows.at[slot, j], acc.at[sdix.at[slot, j, 1]],
                                 ssem, add=True)
                for j in range(gb)
            ]
            for d in descs:
                d.wait()

        def step(bb, base0, g, p, prefetch, gnext, row0):
            pn = 1 - p
            if prefetch:
                load(bb, base0, gnext, pn, row0)
            wait_gathers(p)
            if prefetch:
                fire_gathers(pn)
            scatters(p)

        if not gather:
            for j in range(gb):
                pltpu.sync_copy(vals.at[pl.ds(0, _K)], rows.at[0, j])
        for bi in range(ch):
            b = c * ch + bi
            row0 = b * _NP
            base0 = b * cpb + s * cpt if split else s * cpt
            pltpu.sync_copy(vals.at[pl.ds(row0 + s * nrr, nrr)],
                            acc.at[pl.ds(s * nrr, nrr)])
            plsc.subcore_barrier()

            if gather:
                load(b, base0, 0, 0, row0)
                fire_gathers(0)
                m = (ngrp - 1) // 2

                def dbl(t, carry):
                    g = t * 2
                    step(b, base0, g, 0, True, g + 1, row0)
                    step(b, base0, g + 1, 1, True, g + 2, row0)
                    return carry

                lax.fori_loop(0, m, dbl, 0)
                g0 = 2 * m
                if (ngrp - 1) % 2 == 1:
                    step(b, base0, g0, 0, True, g0 + 1, row0)
                    step(b, base0, g0 + 1, 1, False, 0, row0)
                else:
                    step(b, base0, g0, 0, False, 0, row0)
            else:

                def grp(g, carry):
                    load(b, base0, g, 0, row0)
                    scatters(0)
                    return carry

                lax.fori_loop(0, ngrp, grp, 0)
            plsc.subcore_barrier()
            pltpu.sync_copy(acc.at[pl.ds(s * nrr, nrr)],
                            out.at[pl.ds(row0 + s * nrr, nrr)])
            plsc.subcore_barrier()

    return pl.kernel(
        body,
        out_type=jax.ShapeDtypeStruct((nb * _NP, w), jnp.float32),
        mesh=mesh,
        scratch_types=[
            pltpu.VMEM_SHARED((_NP, w), jnp.float32),
            pltpu.VMEM((2, gb, 2, _K), jnp.int32),
            pltpu.VMEM((2, gb, _K, w), jnp.float32),
            pltpu.SemaphoreType.DMA,
            pltpu.SemaphoreType.DMA,
        ],
        compiler_params=pltpu.CompilerParams(use_tc_tiling_on_sc=False),
    )


def _tc_run(body, row_ins, bcast_ins, out_cols):
    grid = (_NP // _BN,)
    in_specs = (
        [pl.BlockSpec((_BN, a.shape[1]), lambda i: (i, 0)) for a in row_ins]
        + [pl.BlockSpec(w.shape, lambda i: (0, 0)) for w in bcast_ins]
    )
    out_specs = [pl.BlockSpec((_BN, c), lambda i: (i, 0)) for c in out_cols]
    out_shape = [jax.ShapeDtypeStruct((_NP, c), jnp.float32) for c in out_cols]
    res = pl.pallas_call(
        body, grid=grid, in_specs=in_specs, out_specs=out_specs,
        out_shape=out_shape,
    )(*row_ins, *bcast_ins)
    return res


def _row_ids(cols):
    return (pl.program_id(0) * _BN
            + lax.broadcasted_iota(jnp.int32, (_BN, cols), 0))


def _t1_body(x_r, du_r, dv_r, da_r, db_r, diu_r, div_r, diuv_r, xsu_r, xsv_r):
    x = x_r[...]
    diu = 1.0 / jnp.sqrt(du_r[:, 0:1])
    div = 1.0 / jnp.sqrt(dv_r[:, 0:1])
    diuv = 1.0 / jnp.sqrt(da_r[:, 0:1] + db_r[:, 0:1] - 1.0)
    diu_r[...] = jnp.broadcast_to(diu, (_BN, 16))
    div_r[...] = jnp.broadcast_to(div, (_BN, 16))
    diuv_r[...] = jnp.broadcast_to(diuv, (_BN, 16))
    xsu_r[...] = jnp.broadcast_to(x * diu, (_BN, 16))
    xsv_r[...] = jnp.broadcast_to(x * div, (_BN, 16))


def _t2_body(aggu_r, aggv_r, diu_r, div_r, diuv_r, wu_r, bu_r, wv_r, bv_r,
             xs_r):
    yu = bu_r[...] + (diu_r[:, 0:1] * aggu_r[:, 0:1]) * wu_r[...]
    yv = bv_r[...] + (div_r[:, 0:1] * aggv_r[:, 0:1]) * wv_r[...]
    y = jnp.where(_row_ids(64) < _NU, yu, yv)
    xs_r[...] = diuv_r[:, 0:1] * y


def _t3_body(a0_r, a1_r, diu_r, div_r, diuv_r, wuv_r, buv_r, wu2_r, wv2_r,
             xsu_r, xsv_r):
    agg = jnp.concatenate([a0_r[...], a1_r[...]], axis=1)
    x2 = jax.nn.relu(
        buv_r[...]
        + jnp.dot(diuv_r[:, 0:1] * agg, wuv_r[...],
                  preferred_element_type=jnp.float32))
    xsu_r[...] = diu_r[:, 0:1] * jnp.dot(x2, wu2_r[...],
                                         preferred_element_type=jnp.float32)
    xsv_r[...] = div_r[:, 0:1] * jnp.dot(x2, wv2_r[...],
                                         preferred_element_type=jnp.float32)


def _t4_body(a0_r, a1_r, diu_r, div_r, diuv_r, bu_r, bv_r, xs_r):
    outu = bu_r[...] + diu_r[:, 0:1] * a0_r[...]
    outv = bv_r[...] + div_r[:, 0:1] * a1_r[...]
    y = jnp.where(_row_ids(32) < _NU, outu, outv)
    xs_r[...] = diuv_r[:, 0:1] * y


def _t5_body(a0_r, a1_r, diu_r, div_r, diuv_r, wuv_r, buv_r, xsu_r, xsv_r):
    agg = jnp.concatenate([a0_r[...], a1_r[...]], axis=1)
    x4 = jax.nn.relu(
        buv_r[...]
        + jnp.dot(diuv_r[:, 0:1] * agg, wuv_r[...],
                  preferred_element_type=jnp.float32))
    xsu_r[...] = diu_r[:, 0:1] * x4
    xsv_r[...] = div_r[:, 0:1] * x4


def _t6_body(a0_r, a1_r, diu_r, div_r, diuv_r, wu_r, bu_r, wv_r, bv_r, xs_r):
    tu = bu_r[...] + jnp.dot(diu_r[:, 0:1] * a0_r[...], wu_r[...],
                             preferred_element_type=jnp.float32)
    tv = bv_r[...] + jnp.dot(div_r[:, 0:1] * a1_r[...], wv_r[...],
                             preferred_element_type=jnp.float32)
    t = jnp.where(_row_ids(128) < _NU, tu, tv)
    xs_r[...] = diuv_r[:, 0:1] * t


def _t7_body(a0_r, a1_r, a2_r, a3_r, diuv_r, wuv_r, buv_r, out_r):
    agg = jnp.concatenate([a0_r[...], a1_r[...], a2_r[...], a3_r[...]],
                          axis=1)
    out_r[...] = buv_r[...] + jnp.dot(
        diuv_r[:, 0:1] * agg, wuv_r[...], preferred_element_type=jnp.float32)


def kernel(x, edge_index, edge_index_u, edge_index_v, params):
    f32 = jnp.float32
    ((Wu1, bu1, Wv1, bv1, Wuv1, buv1),
     (Wu2, bu2, Wv2, bv2, Wuv2, buv2),
     (Wu3, bu3, Wv3, bv3, Wuv3, buv3)) = params

    src_uv, dst_uv = edge_index[0], edge_index[1]
    src_u, dst_u = edge_index_u[0], edge_index_u[1]
    src_v, dst_v = edge_index_v[0], edge_index_v[1]

    def pad_e(a, val, ep):
        return jnp.pad(a, (0, ep - a.shape[0]), constant_values=val)

    def mk_sd(src_blocks, dst_blocks, ep):
        srcs = jnp.stack(src_blocks).reshape(len(src_blocks), ep // _K, _K)
        dsts = jnp.stack(dst_blocks).reshape(len(dst_blocks), ep // _K, _K)
        return jnp.stack([srcs, dsts], axis=2)

    spu = pad_e(src_u, 0, _EP)
    spv = pad_e(src_v, 0, _EP)
    spuv = pad_e(src_uv, 0, _EP)
    dpu = pad_e(dst_u, _N, _EP)
    dpv = pad_e(dst_v, _N, _EP)
    dpuv = pad_e(dst_uv, _N, _EP)

    sd_ace = mk_sd([spu, spv], [dpu, dpv], _EP)
    sd_uv = mk_sd([spuv], [dpuv], _EP)

    seg16 = _make_seg_sum(2, 16, _EP)
    seg16s = _make_seg_sum(2, 16, _EP, shared=True)
    seg32 = _make_seg_sum(2, 32, _EP)
    seg32s = _make_seg_sum(2, 32, _EP, shared=True)
    seg32x4s = _make_seg_sum(4, 32, _EP, shared=True)
    cnt16 = _make_seg_sum(2, 16, _EP, gather=False)
    cnt16sp = _make_seg_sum(2, 16, _EP, gather=False, split=True)

    ones2 = jnp.ones((2 * _NP, 16), f32)
    h1 = cnt16(ones2, sd_ace)
    h2 = cnt16sp(ones2, sd_uv)

    xpad = jnp.pad(x, ((0, _NP - _N), (0, 0)))
    diu, div, diuv, xsu, xsv = _tc_run(
        _t1_body, [xpad, h1[:_NP], h1[_NP:], h2[:_NP], h2[_NP:]], [],
        [16, 16, 16, 16, 16])

    agg_a = seg16(jnp.concatenate([xsu, xsv]), sd_ace)
    xs_b = _tc_run(
        _t2_body, [agg_a[:_NP], agg_a[_NP:], diu, div, diuv],
        [Wu1.reshape(1, 64), bu1.reshape(1, 64),
         Wv1.reshape(1, 64), bv1.reshape(1, 64)], [64])[0]
    agg_b = seg32s(jnp.concatenate([xs_b[:, :32], xs_b[:, 32:]]), sd_uv)

    xsu2, xsv2 = _tc_run(
        _t3_body, [agg_b[:_NP], agg_b[_NP:], diu, div, diuv],
        [Wuv1, buv1.reshape(1, 64), Wu2, Wv2], [32, 32])
    agg_c = seg32(jnp.concatenate([xsu2, xsv2]), sd_ace)
    xs_d = _tc_run(
        _t4_body, [agg_c[:_NP], agg_c[_NP:], diu, div, diuv],
        [bu2.reshape(1, 32), bv2.reshape(1, 32)], [32])[0]
    agg_d = seg16s(jnp.concatenate([xs_d[:, :16], xs_d[:, 16:]]), sd_uv)

    xsu3, xsv3 = _tc_run(
        _t5_body, [agg_d[:_NP], agg_d[_NP:], diu, div, diuv],
        [Wuv2, buv2.reshape(1, 32)], [32, 32])
    agg_e = seg32(jnp.concatenate([xsu3, xsv3]), sd_ace)
    xs_f = _tc_run(
        _t6_body, [agg_e[:_NP], agg_e[_NP:], diu, div, diuv],
        [Wu3, bu3.reshape(1, 128), Wv3, bv3.reshape(1, 128)], [128])[0]
    agg_f = seg32x4s(
        jnp.concatenate([xs_f[:, 0:32], xs_f[:, 32:64],
                         xs_f[:, 64:96], xs_f[:, 96:128]]), sd_uv)
    out = _tc_run(
        _t7_body,
        [agg_f[:_NP], agg_f[_NP:2 * _NP], agg_f[2 * _NP:3 * _NP],
         agg_f[3 * _NP:], diuv],
        [Wuv3, buv3.reshape(1, 128)], [128])[0]
    return out[:_N]
